# baseline (speedup 1.0000x reference)
import jax
import jax.numpy as jnp
from jax import lax
from jax.experimental import pallas as pl
from jax.experimental.pallas import tpu as pltpu
from jax.experimental.pallas import tpu_sc as plsc

B = 256
L = 77
D = 1024
NVEC = 8

NC = 2
NS = 16
N = B * L
NEVEN = ((L + 1) // 2) * B
CHUNK = 32
NMAIN = 19
SLOTS = 4 * B


def _sc_kernel(ids_hbm, table_hbm, tisrc_hbm, didx_hbm, meta_hbm, out_hbm,
               ids_v, buf0, buf1, ti_a, ti_b, meta_v, didx_v,
               gsem0, gsem1, wsem0, wsem1, dsem):
    cid = lax.axis_index("c")
    sid = lax.axis_index("s")
    bufs = (buf0, buf1)
    gsems = (gsem0, gsem1)
    wsems = (wsem0, wsem1)

    ids_off = NEVEN * cid + (NMAIN * CHUNK) * sid
    pltpu.sync_copy(ids_hbm.at[pl.ds(ids_off, NMAIN * CHUNK)],
                    ids_v.at[pl.ds(0, NMAIN * CHUNK)])
    pltpu.sync_copy(tisrc_hbm.at[cid, 0], ti_a)
    pltpu.sync_copy(tisrc_hbm.at[cid, 1], ti_b)

    blk0 = NMAIN * sid

    def out_row(bg):
        return 512 * (bg // 8) + 256 * cid + 32 * (bg % 8)

    def gather(k):
        return pltpu.async_copy(
            table_hbm.at[ids_v.at[pl.ds(k * CHUNK, CHUNK)]],
            bufs[k % 2], gsems[k % 2])

    def writeback(k):
        return pltpu.async_copy(
            bufs[k % 2], out_hbm.at[pl.ds(out_row(blk0 + k), CHUNK)],
            wsems[k % 2])

    g = [None] * NMAIN
    w = [None] * NMAIN
    for k in range(NMAIN):
        if k >= 2:
            w[k - 2].wait()
        g[k] = gather(k)
        if k >= 1:
            g[k - 1].wait()
            w[k - 1] = writeback(k - 1)
    g[NMAIN - 1].wait()
    w[NMAIN - 1] = writeback(NMAIN - 1)
    w[NMAIN - 2].wait()
    w[NMAIN - 1].wait()

    @pl.when(jnp.logical_and(cid == 0, sid < 8))
    def _():
        ebg = 16 * NMAIN + sid
        pltpu.sync_copy(ids_hbm.at[pl.ds(ebg * CHUNK, CHUNK)],
                        ids_v.at[pl.ds(NMAIN * CHUNK, CHUNK)])
        pltpu.async_copy(
            table_hbm.at[ids_v.at[pl.ds(NMAIN * CHUNK, CHUNK)]],
            buf0, gsem0).wait()
        pltpu.sync_copy(buf0, out_hbm.at[pl.ds(out_row(ebg), CHUNK)])

    plsc.subcore_barrier()

    tis = (ti_a, ti_b)
    for part in range(2):
        pltpu.sync_copy(meta_hbm.at[cid, part], meta_v)
        ngroups = jnp.max(meta_v[...])
        nmine = jnp.where(ngroups > sid, (ngroups - sid - 1) // NS + 1, 0)

        def it(i, carry, part=part):
            grp = sid + i * NS
            pltpu.sync_copy(didx_hbm.at[cid, part, pl.ds(grp * 16, 16)],
                            didx_v)
            pltpu.async_copy(tis[part], out_hbm.at[didx_v], dsem).wait()
            return carry

        lax.fori_loop(0, nmine, it, 0)


@jax.jit
def kernel(input_ids, table, ti_emb, offsets):
    idsT = input_ids.T
    ids_perm = jnp.concatenate(
        [idsT[0::2].reshape(-1), idsT[1::2].reshape(-1)])

    par = (offsets & 1).astype(jnp.int32)
    in_part = jnp.stack([1 - par, par])
    nprompts = jnp.sum(in_part, axis=1)
    rank = jnp.cumsum(in_part, axis=1) - 1
    jj = jnp.arange(4, dtype=jnp.int32)
    slot_idx = jnp.arange(SLOTS, dtype=jnp.int32)
    bvec = jnp.arange(B, dtype=jnp.int32)

    didx = []
    tisrc = []
    for c in range(NC):
        row_d = []
        row_s = []
        for part in range(2):
            jset = 2 * jj + ((1 - c) if part == 0 else c)
            dest = (offsets[:, None] + 1 + jset[None, :]) * B + bvec[:, None]
            slots = rank[part][:, None] * 4 + jj[None, :]
            valid = in_part[part].astype(bool)
            scat = jnp.zeros((SLOTS,), jnp.int32).at[
                jnp.where(valid[:, None], slots, SLOTS)
            ].set(dest, mode="drop")
            full = jnp.where(slot_idx < 4 * nprompts[part],
                             scat, scat[slot_idx % 4])
            row_d.append(full)
            row_s.append(jnp.tile(ti_emb[jset], (4, 1)))
        didx.append(jnp.stack(row_d))
        tisrc.append(jnp.stack(row_s))
    didx = jnp.stack(didx)
    tisrc = jnp.stack(tisrc)
    ngroups = (nprompts + 3) // 4
    meta = jnp.broadcast_to(ngroups[None, :, None], (NC, 2, 16)).astype(
        jnp.int32)

    mesh = plsc.VectorSubcoreMesh(core_axis_name="c", subcore_axis_name="s")
    out2 = pl.kernel(
        _sc_kernel,
        out_type=jax.ShapeDtypeStruct((N, D), jnp.float32),
        mesh=mesh,
        scratch_types=(
            [pltpu.VMEM(((NMAIN + 1) * CHUNK,), jnp.int32)]
            + [pltpu.VMEM((CHUNK, D), jnp.float32)] * 2
            + [pltpu.VMEM((16, D), jnp.float32)] * 2
            + [pltpu.VMEM((16,), jnp.int32)] * 2
            + [pltpu.SemaphoreType.DMA] * 5
        ),
        compiler_params=pltpu.CompilerParams(needs_layout_passes=False),
    )(ids_perm, table, tisrc, didx, meta)
    return out2.reshape(L, B, D).transpose(1, 0, 2)


# speedup vs baseline: 1.6207x; 1.6207x over previous
import jax
import jax.numpy as jnp
from jax import lax
from jax.experimental import pallas as pl
from jax.experimental.pallas import tpu as pltpu
from jax.experimental.pallas import tpu_sc as plsc

VOCAB = 49408
B = 256
L = 77
D = 1024
NVEC = 8

NC = 2
NS = 16
NW = NC * NS
N = B * L
PER_W = N // NW
BPW = B // NW
CHUNK = 56
NBUF = 2
NFULL = PER_W // CHUNK
TAIL = PER_W - NFULL * CHUNK


def _gather_kernel(ids_hbm, table_hbm, out_hbm, ids_v, *bufs_and_sems):
    bufs = bufs_and_sems[:NBUF]
    gsems = bufs_and_sems[NBUF:2 * NBUF]
    wsems = bufs_and_sems[2 * NBUF:3 * NBUF]
    wid = lax.axis_index("s") * NC + lax.axis_index("c")
    base = wid * PER_W

    pltpu.sync_copy(ids_hbm.at[pl.ds(base, PER_W)], ids_v)

    sizes = [CHUNK] * NFULL + ([TAIL] if TAIL else [])
    nchunks = len(sizes)

    def gather(c):
        return pltpu.async_copy(
            table_hbm.at[ids_v.at[pl.ds(c * CHUNK, sizes[c])]],
            bufs[c % NBUF].at[pl.ds(0, sizes[c])], gsems[c % NBUF])

    def writeback(c):
        return pltpu.async_copy(
            bufs[c % NBUF].at[pl.ds(0, sizes[c])],
            out_hbm.at[pl.ds(base + c * CHUNK, sizes[c])], wsems[c % NBUF])

    g = [None] * nchunks
    w = [None] * nchunks
    for c in range(nchunks):
        if c >= NBUF:
            w[c - NBUF].wait()
        g[c] = gather(c)
        if c >= 1:
            g[c - 1].wait()
            w[c - 1] = writeback(c - 1)
    g[nchunks - 1].wait()
    w[nchunks - 1] = writeback(nchunks - 1)
    for c in range(max(0, nchunks - NBUF), nchunks - 1):
        w[c].wait()
    w[nchunks - 1].wait()


def _ti_kernel(ti2_hbm, dest_hbm, out_ref,
               ti_v, didx0, didx1, didx2, didx3,
               isem0, isem1, isem2, isem3, dsem):
    wid = lax.axis_index("s") * NC + lax.axis_index("c")
    didxs = (didx0, didx1, didx2, didx3)
    isems = (isem0, isem1, isem2, isem3)
    ic = [pltpu.async_copy(dest_hbm.at[wid, g], didxs[g], isems[g])
          for g in range(BPW // 2)]
    pltpu.sync_copy(ti2_hbm, ti_v)
    scat = []
    for g in range(BPW // 2):
        ic[g].wait()
        scat.append(pltpu.async_copy(ti_v, out_ref.at[didxs[g]], dsem))
    for s in scat:
        s.wait()


@jax.jit
def kernel(input_ids, table, ti_emb, offsets):
    ids_lm = input_ids.T.reshape(N)
    ti2 = jnp.concatenate([ti_emb, ti_emb], axis=0)
    lane = jnp.arange(16, dtype=jnp.int32)
    grp = jnp.arange(BPW // 2, dtype=jnp.int32)
    b = (jnp.arange(NW, dtype=jnp.int32)[:, None, None] * BPW
         + grp[None, :, None] * 2 + (lane[None, None, :] >> 3))
    dest = (offsets[b] + 1 + (lane[None, None, :] & 7)) * B + b

    mesh = plsc.VectorSubcoreMesh(core_axis_name="c", subcore_axis_name="s")
    out2 = pl.kernel(
        _gather_kernel,
        out_type=jax.ShapeDtypeStruct((N, D), jnp.float32),
        mesh=mesh,
        scratch_types=(
            [pltpu.VMEM((PER_W,), jnp.int32)]
            + [pltpu.VMEM((CHUNK, D), jnp.float32)] * NBUF
            + [pltpu.SemaphoreType.DMA] * (2 * NBUF)
        ),
    )(ids_lm, table)

    out_ref = jax.new_ref(out2)
    pl.kernel(
        _ti_kernel,
        out_type=(),
        mesh=mesh,
        scratch_types=(
            [pltpu.VMEM((16, D), jnp.float32)]
            + [pltpu.VMEM((16,), jnp.int32)] * (BPW // 2)
            + [pltpu.SemaphoreType.DMA] * (BPW // 2 + 1)
        ),
    )(ti2, dest, out_ref)
    out = jax.freeze(out_ref)
    return out.reshape(L, B, D).transpose(1, 0, 2)
